# baseline (speedup 1.0000x reference)
import functools

import jax
import jax.numpy as jnp
from jax import lax
from jax.experimental import pallas as pl
from jax.experimental.pallas import tpu as pltpu
from jax.experimental.pallas import tpu_sc as plsc

N_BOX = 8732
N_PAD = 8736
N_SPAD = 8960
N_M1 = 768
N_M2 = 48
TOP_K = 200
SEL_PAD = 224
N_CLS = 21
N_IMG = 8
CONF_T = 0.01
NMS_T = 0.45
NEG = -2.0


def _prep_body(conf_ref, loc_ref, dbox_ref, scores_ref, boxes_ref, area_ref):
    c = conf_ref[0]
    m = jnp.max(c, axis=0, keepdims=True)
    e = jnp.exp(c - m)
    den = e[0:1]
    for i in range(1, N_CLS):
        den = den + e[i:i + 1]
    p = e / den
    s = jnp.where(p > CONF_T, p, -1.0)
    spad = jnp.full((N_CLS, N_PAD - N_BOX), NEG, jnp.float32)
    scores_ref[0] = jnp.concatenate([s, spad], axis=1)

    l = loc_ref[0]
    d = dbox_ref[...]
    cx = d[0:1] + d[0:1] * 0.1 * l[0:1]
    cy = d[1:2] + d[1:2] * 0.1 * l[1:2]
    sw = d[2:3] * jnp.exp(l[2:3] * 0.2)
    sh = d[3:4] * jnp.exp(l[3:4] * 0.2)
    x1 = cx - sw / 2.0
    y1 = cy - sh / 2.0
    x2 = x1 + sw
    y2 = y1 + sh
    bx = jnp.concatenate([x1, y1, x2, y2], axis=0)
    bpad = jnp.zeros((4, N_PAD - N_BOX), jnp.float32)
    boxes_ref[0] = jnp.concatenate([bx, bpad], axis=1)
    ar = (x2 - x1) * (y2 - y1)
    apad = jnp.zeros((1, N_PAD - N_BOX), jnp.float32)
    area_ref[0] = jnp.concatenate([ar, apad], axis=1)


_prep = pl.pallas_call(
    _prep_body,
    grid=(N_IMG,),
    in_specs=[
        pl.BlockSpec((1, N_CLS, N_BOX), lambda i: (i, 0, 0)),
        pl.BlockSpec((1, 4, N_BOX), lambda i: (i, 0, 0)),
        pl.BlockSpec((4, N_BOX), lambda i: (0, 0)),
    ],
    out_specs=[
        pl.BlockSpec((1, N_CLS, N_PAD), lambda i: (i, 0, 0)),
        pl.BlockSpec((1, 4, N_PAD), lambda i: (i, 0, 0)),
        pl.BlockSpec((1, 1, N_PAD), lambda i: (i, 0, 0)),
    ],
    out_shape=[
        jax.ShapeDtypeStruct((N_IMG, N_CLS, N_PAD), jnp.float32),
        jax.ShapeDtypeStruct((N_IMG, 4, N_PAD), jnp.float32),
        jax.ShapeDtypeStruct((N_IMG, 1, N_PAD), jnp.float32),
    ],
)


_MESH = plsc.VectorSubcoreMesh(
    core_axis_name="c", subcore_axis_name="s", num_cores=2, num_subcores=16)


@functools.partial(
    pl.kernel,
    out_type=jax.ShapeDtypeStruct((N_IMG, N_CLS, TOP_K * 5), jnp.float32),
    mesh=_MESH,
    scratch_types=[
        pltpu.VMEM((N_SPAD,), jnp.float32),
        pltpu.VMEM((N_PAD,), jnp.float32),
        pltpu.VMEM((N_PAD,), jnp.float32),
        pltpu.VMEM((N_PAD,), jnp.float32),
        pltpu.VMEM((N_PAD,), jnp.float32),
        pltpu.VMEM((N_PAD,), jnp.float32),
        pltpu.VMEM((N_M1,), jnp.float32),
        pltpu.VMEM((N_M2,), jnp.float32),
        pltpu.VMEM((SEL_PAD,), jnp.int32),
        pltpu.VMEM((SEL_PAD,), jnp.float32),
        pltpu.VMEM((SEL_PAD,), jnp.float32),
        pltpu.VMEM((SEL_PAD,), jnp.float32),
        pltpu.VMEM((SEL_PAD,), jnp.float32),
        pltpu.VMEM((SEL_PAD,), jnp.float32),
        pltpu.VMEM((SEL_PAD,), jnp.float32),
        pltpu.VMEM((SEL_PAD,), jnp.int32),
        pltpu.VMEM((63 * 16,), jnp.float32),
    ],
    compiler_params=pltpu.CompilerParams(
        needs_layout_passes=False, use_tc_tiling_on_sc=False),
)
def _sc_nms(scores_hbm, boxes_hbm, area_hbm, out_hbm,
            s_v, bx1_v, by1_v, bx2_v, by2_v, ar_v, m1_v, m2_v,
            si_v, ss_v, sx1_v, sy1_v, sx2_v, sy2_v, sar_v, sup_v, ob_v):
    wid = lax.axis_index("c") * 16 + lax.axis_index("s")
    b = wid // 4
    part = wid % 4
    iota = lax.iota(jnp.int32, 16)
    zero16f = jnp.zeros((16,), jnp.float32)
    zero16i = jnp.zeros((16,), jnp.int32)
    negv = jnp.full((16,), NEG, jnp.float32)
    mask0 = iota == 0
    rot8 = (iota + 8) & 15
    rot4 = (iota + 4) & 15
    rot2 = (iota + 2) & 15
    rot1 = (iota + 1) & 15
    _dn = lax.GatherDimensionNumbers(
        offset_dims=(), collapsed_slice_dims=(0,), start_index_map=(0,))

    def perm(v, idx):
        return lax.gather(v, idx[:, None], _dn, slice_sizes=(1,),
                          mode=lax.GatherScatterMode.PROMISE_IN_BOUNDS)

    def treemax(v):
        v = jnp.maximum(v, perm(v, rot8))
        v = jnp.maximum(v, perm(v, rot4))
        v = jnp.maximum(v, perm(v, rot2))
        v = jnp.maximum(v, perm(v, rot1))
        return v

    def lastfind(v, m, base):
        f = plsc.all_reduce_ffs(lax.rev(v, (0,)) == m)
        return base + 15 - f, f < 16

    pltpu.sync_copy(boxes_hbm.at[b, 0], bx1_v)
    pltpu.sync_copy(boxes_hbm.at[b, 1], by1_v)
    pltpu.sync_copy(boxes_hbm.at[b, 2], bx2_v)
    pltpu.sync_copy(boxes_hbm.at[b, 3], by2_v)
    pltpu.sync_copy(area_hbm.at[b, 0], ar_v)

    def _pad_s(i, _):
        s_v[pl.ds(N_PAD + i * 16, 16)] = negv
        return 0
    lax.fori_loop(0, (N_SPAD - N_PAD) // 16, _pad_s, 0)

    def _pad_m1(i, _):
        m1_v[pl.ds(560 + i * 16, 16)] = negv
        return 0
    lax.fori_loop(0, (N_M1 - 560) // 16, _pad_m1, 0)

    def _zero_ob0(i, _):
        ob_v[pl.ds(i * 16, 16)] = zero16f
        return 0
    lax.fori_loop(0, 63, _zero_ob0, 0)

    @pl.when(wid < N_IMG)
    def _():
        pltpu.sync_copy(ob_v.at[pl.ds(0, TOP_K * 5)], out_hbm.at[wid, 0])

    def _pair(k, _):
        cidx = 1 + part * 5 + k
        pltpu.sync_copy(scores_hbm.at[b, cidx], s_v.at[pl.ds(0, N_PAD)])

        def m1_body(j, _):
            base = iota * 16 + j * 256
            acc = plsc.load_gather(s_v, [base])
            for t in range(1, 16):
                acc = jnp.maximum(acc, plsc.load_gather(s_v, [base + t]))
            m1_v[pl.ds(j * 16, 16)] = acc
            return 0
        lax.fori_loop(0, 35, m1_body, 0)

        def m2_body(j, _):
            base = iota * 16 + j * 256
            acc = plsc.load_gather(m1_v, [base])
            for t in range(1, 16):
                acc = jnp.maximum(acc, plsc.load_gather(m1_v, [base + t]))
            m2_v[pl.ds(j * 16, 16)] = acc
            return 0
        lax.fori_loop(0, 3, m2_body, 0)

        w0 = m2_v[pl.ds(0, 16)]
        w1 = m2_v[pl.ds(16, 16)]
        w2 = m2_v[pl.ds(32, 16)]
        m0 = treemax(jnp.maximum(jnp.maximum(w0, w1), w2))

        def ext_body(j, carry):
            m, w0, w1, w2 = carry
            s2, h2 = lastfind(w2, m, 32)
            s1, h1 = lastfind(w1, m, 16)
            s0, _ = lastfind(w0, m, 0)
            sci = jnp.where(h2, s2, jnp.where(h1, s1, s0))
            mv = plsc.load_gather(m1_v, [sci * 16 + iota])
            ci, _ = lastfind(mv, m, sci * 16)
            ev = plsc.load_gather(s_v, [ci * 16 + iota])
            li, _ = lastfind(ev, m, 0)
            gidx = ci * 16 + li
            jv = jnp.full((16,), j, jnp.int32)
            plsc.store_scatter(si_v, [jv], gidx, mask=mask0)
            plsc.store_scatter(ss_v, [jv], m, mask=mask0)
            ev2 = jnp.where(iota == li, NEG, ev)
            plsc.store_scatter(s_v, [gidx], negv, mask=mask0)
            nm1 = treemax(ev2)
            plsc.store_scatter(m1_v, [ci], nm1, mask=mask0)
            mvo = jnp.where(iota == (ci - sci * 16), NEG, mv)
            nm2 = jnp.maximum(treemax(mvo), nm1)
            upd = iota == (sci & 15)
            wsel = sci >> 4
            u0 = (wsel == 0) & upd
            u1 = (wsel == 1) & upd
            u2 = (wsel == 2) & upd
            om = treemax(jnp.maximum(jnp.maximum(
                jnp.where(u0, NEG, w0), jnp.where(u1, NEG, w1)),
                jnp.where(u2, NEG, w2)))
            nm = jnp.maximum(om, nm2)
            n0 = jnp.where(u0, nm2, w0)
            n1 = jnp.where(u1, nm2, w1)
            n2 = jnp.where(u2, nm2, w2)
            return nm, n0, n1, n2
        lax.fori_loop(0, TOP_K, ext_body, (m0, w0, w1, w2))

        tmask = (iota + 192) >= TOP_K
        tsl = pl.ds(192, 16)
        si_v[tsl] = jnp.where(tmask, 0, si_v[tsl])
        ss_v[tsl] = jnp.where(tmask, -1.0, ss_v[tsl])
        si_v[pl.ds(208, 16)] = zero16i
        ss_v[pl.ds(208, 16)] = jnp.full((16,), -1.0, jnp.float32)

        def gsel(g, _):
            sl = pl.ds(g * 16, 16)
            idxv = si_v[sl]
            sx1_v[sl] = plsc.load_gather(bx1_v, [idxv])
            sy1_v[sl] = plsc.load_gather(by1_v, [idxv])
            sx2_v[sl] = plsc.load_gather(bx2_v, [idxv])
            sy2_v[sl] = plsc.load_gather(by2_v, [idxv])
            sar_v[sl] = plsc.load_gather(ar_v, [idxv])
            sup_v[sl] = zero16i
            return 0
        lax.fori_loop(0, 14, gsel, 0)

        def nms_body(j, _):
            jv = jnp.full((16,), j, jnp.int32)
            scj = plsc.load_gather(ss_v, [jv])
            suj = plsc.load_gather(sup_v, [jv])
            kv = ((scj > CONF_T) & (suj == 0)).astype(jnp.int32)

            @pl.when(kv[0] == 1)
            def _():
                x1p = plsc.load_gather(sx1_v, [jv])
                y1p = plsc.load_gather(sy1_v, [jv])
                x2p = plsc.load_gather(sx2_v, [jv])
                y2p = plsc.load_gather(sy2_v, [jv])
                arp = plsc.load_gather(sar_v, [jv])

                def iou_cond(sl, gbase):
                    tx1 = jnp.maximum(sx1_v[sl], x1p)
                    ty1 = jnp.maximum(sy1_v[sl], y1p)
                    tx2 = jnp.minimum(sx2_v[sl], x2p)
                    ty2 = jnp.minimum(sy2_v[sl], y2p)
                    w = jnp.maximum(tx2 - tx1, 0.0)
                    h = jnp.maximum(ty2 - ty1, 0.0)
                    inter = w * h
                    union = arp + (sar_v[sl] - inter)
                    iou = inter / union
                    return (iou > NMS_T) & ((iota + gbase) > j)

                def inner(g2, _):
                    g = g2 * 2
                    sl0 = pl.ds(g * 16, 16)
                    sl1 = pl.ds(g * 16 + 16, 16)
                    c0 = iou_cond(sl0, g * 16)
                    c1 = iou_cond(sl1, g * 16 + 16)
                    s0 = sup_v[sl0]
                    s1 = sup_v[sl1]
                    sup_v[sl0] = jnp.where(c0, 1, s0)
                    sup_v[sl1] = jnp.where(c1, 1, s1)
                    return 0
                lax.fori_loop(j // 32, 7, inner, 0)
            return 0
        lax.fori_loop(0, TOP_K, nms_body, 0)

        def zero_ob(i, _):
            ob_v[pl.ds(i * 16, 16)] = zero16f
            return 0
        lax.fori_loop(0, 63, zero_ob, 0)

        def out_body(g, off):
            sl = pl.ds(g * 16, 16)
            scv = ss_v[sl]
            keep = (scv > CONF_T) & (sup_v[sl] == 0)
            ki = keep.astype(jnp.int32)
            pos = off + jnp.cumsum(ki) - 1
            base5 = pos * 5
            plsc.store_scatter(ob_v, [base5], scv, mask=keep)
            plsc.store_scatter(ob_v, [base5 + 1], sx1_v[sl], mask=keep)
            plsc.store_scatter(ob_v, [base5 + 2], sy1_v[sl], mask=keep)
            plsc.store_scatter(ob_v, [base5 + 3], sx2_v[sl], mask=keep)
            plsc.store_scatter(ob_v, [base5 + 4], sy2_v[sl], mask=keep)
            return off + jnp.sum(ki)
        lax.fori_loop(0, 13, out_body, jnp.int32(0))

        pltpu.sync_copy(ob_v.at[pl.ds(0, TOP_K * 5)], out_hbm.at[b, cidx])
        return 0

    lax.fori_loop(0, 5, _pair, 0)


def kernel(loc_data, conf_data, dbox_list):
    conf_t = jnp.transpose(conf_data, (0, 2, 1))
    loc_t = jnp.transpose(loc_data, (0, 2, 1))
    dbox_t = jnp.transpose(dbox_list, (1, 0))
    scores, boxes, area = _prep(conf_t, loc_t, dbox_t)
    out = _sc_nms(scores, boxes, area)
    return out.reshape(N_IMG, N_CLS, TOP_K, 5)


# speedup vs baseline: 1.0567x; 1.0567x over previous
import functools

import jax
import jax.numpy as jnp
from jax import lax
from jax.experimental import pallas as pl
from jax.experimental.pallas import tpu as pltpu
from jax.experimental.pallas import tpu_sc as plsc

N_BOX = 8732
N_PAD = 8736
N_SPAD = 8960
N_M1 = 768
N_M2 = 48
TOP_K = 200
SEL_PAD = 224
N_CLS = 21
N_IMG = 8
CONF_T = 0.01
NMS_T = 0.45
NEG = -2.0


def _prep_body(conf_ref, loc_ref, dbox_ref, scores_ref, boxes_ref, area_ref):
    c = conf_ref[0]
    m = jnp.max(c, axis=0, keepdims=True)
    e = jnp.exp(c - m)
    den = e[0:1]
    for i in range(1, N_CLS):
        den = den + e[i:i + 1]
    p = e / den
    s = jnp.where(p > CONF_T, p, -1.0)
    spad = jnp.full((N_CLS, N_PAD - N_BOX), NEG, jnp.float32)
    scores_ref[0] = jnp.concatenate([s, spad], axis=1)

    l = loc_ref[0]
    d = dbox_ref[...]
    cx = d[0:1] + d[0:1] * 0.1 * l[0:1]
    cy = d[1:2] + d[1:2] * 0.1 * l[1:2]
    sw = d[2:3] * jnp.exp(l[2:3] * 0.2)
    sh = d[3:4] * jnp.exp(l[3:4] * 0.2)
    x1 = cx - sw / 2.0
    y1 = cy - sh / 2.0
    x2 = x1 + sw
    y2 = y1 + sh
    bx = jnp.concatenate([x1, y1, x2, y2], axis=0)
    bpad = jnp.zeros((4, N_PAD - N_BOX), jnp.float32)
    boxes_ref[0] = jnp.concatenate([bx, bpad], axis=1)
    ar = (x2 - x1) * (y2 - y1)
    apad = jnp.zeros((1, N_PAD - N_BOX), jnp.float32)
    area_ref[0] = jnp.concatenate([ar, apad], axis=1)


_prep = pl.pallas_call(
    _prep_body,
    grid=(N_IMG,),
    in_specs=[
        pl.BlockSpec((1, N_CLS, N_BOX), lambda i: (i, 0, 0)),
        pl.BlockSpec((1, 4, N_BOX), lambda i: (i, 0, 0)),
        pl.BlockSpec((4, N_BOX), lambda i: (0, 0)),
    ],
    out_specs=[
        pl.BlockSpec((1, N_CLS, N_PAD), lambda i: (i, 0, 0)),
        pl.BlockSpec((1, 4, N_PAD), lambda i: (i, 0, 0)),
        pl.BlockSpec((1, 1, N_PAD), lambda i: (i, 0, 0)),
    ],
    out_shape=[
        jax.ShapeDtypeStruct((N_IMG, N_CLS, N_PAD), jnp.float32),
        jax.ShapeDtypeStruct((N_IMG, 4, N_PAD), jnp.float32),
        jax.ShapeDtypeStruct((N_IMG, 1, N_PAD), jnp.float32),
    ],
)


_MESH = plsc.VectorSubcoreMesh(
    core_axis_name="c", subcore_axis_name="s", num_cores=2, num_subcores=16)


@functools.partial(
    pl.kernel,
    out_type=jax.ShapeDtypeStruct((N_IMG, N_CLS, TOP_K * 5), jnp.float32),
    mesh=_MESH,
    scratch_types=[
        pltpu.VMEM((N_SPAD,), jnp.float32),
        pltpu.VMEM((N_SPAD,), jnp.float32),
        pltpu.VMEM((N_PAD,), jnp.float32),
        pltpu.VMEM((N_PAD,), jnp.float32),
        pltpu.VMEM((N_PAD,), jnp.float32),
        pltpu.VMEM((N_PAD,), jnp.float32),
        pltpu.VMEM((N_PAD,), jnp.float32),
        pltpu.VMEM((N_M1,), jnp.float32),
        pltpu.VMEM((N_M1,), jnp.float32),
        pltpu.VMEM((N_M2,), jnp.float32),
        pltpu.VMEM((N_M2,), jnp.float32),
        pltpu.VMEM((SEL_PAD,), jnp.int32),
        pltpu.VMEM((SEL_PAD,), jnp.float32),
        pltpu.VMEM((SEL_PAD,), jnp.int32),
        pltpu.VMEM((SEL_PAD,), jnp.float32),
        pltpu.VMEM((SEL_PAD,), jnp.float32),
        pltpu.VMEM((SEL_PAD,), jnp.float32),
        pltpu.VMEM((SEL_PAD,), jnp.float32),
        pltpu.VMEM((SEL_PAD,), jnp.float32),
        pltpu.VMEM((SEL_PAD,), jnp.float32),
        pltpu.VMEM((SEL_PAD,), jnp.int32),
        pltpu.VMEM((63 * 16,), jnp.float32),
    ],
    compiler_params=pltpu.CompilerParams(
        needs_layout_passes=False, use_tc_tiling_on_sc=False),
)
def _sc_nms(scores_hbm, boxes_hbm, area_hbm, out_hbm,
            sA, sB, bx1_v, by1_v, bx2_v, by2_v, ar_v,
            m1A, m1B, m2A, m2B, siA, ssA, siB, ssB,
            sx1_v, sy1_v, sx2_v, sy2_v, sar_v, sup_v, ob_v):
    wid = lax.axis_index("c") * 16 + lax.axis_index("s")
    b = wid // 4
    part = wid % 4
    iota = lax.iota(jnp.int32, 16)
    zero16f = jnp.zeros((16,), jnp.float32)
    zero16i = jnp.zeros((16,), jnp.int32)
    negv = jnp.full((16,), NEG, jnp.float32)
    mask0 = iota == 0
    rot8 = (iota + 8) & 15
    rot4 = (iota + 4) & 15
    rot2 = (iota + 2) & 15
    rot1 = (iota + 1) & 15
    _dn = lax.GatherDimensionNumbers(
        offset_dims=(), collapsed_slice_dims=(0,), start_index_map=(0,))

    def perm(v, idx):
        return lax.gather(v, idx[:, None], _dn, slice_sizes=(1,),
                          mode=lax.GatherScatterMode.PROMISE_IN_BOUNDS)

    def treemax(v):
        v = jnp.maximum(v, perm(v, rot8))
        v = jnp.maximum(v, perm(v, rot4))
        v = jnp.maximum(v, perm(v, rot2))
        v = jnp.maximum(v, perm(v, rot1))
        return v

    def lastfind(v, m, base):
        f = plsc.all_reduce_ffs(lax.rev(v, (0,)) == m)
        return base + 15 - f, f < 16

    pltpu.sync_copy(boxes_hbm.at[b, 0], bx1_v)
    pltpu.sync_copy(boxes_hbm.at[b, 1], by1_v)
    pltpu.sync_copy(boxes_hbm.at[b, 2], bx2_v)
    pltpu.sync_copy(boxes_hbm.at[b, 3], by2_v)
    pltpu.sync_copy(area_hbm.at[b, 0], ar_v)

    def _pad_s(i, _):
        sA[pl.ds(N_PAD + i * 16, 16)] = negv
        sB[pl.ds(N_PAD + i * 16, 16)] = negv
        return 0
    lax.fori_loop(0, (N_SPAD - N_PAD) // 16, _pad_s, 0)

    def _pad_m1(i, _):
        m1A[pl.ds(560 + i * 16, 16)] = negv
        m1B[pl.ds(560 + i * 16, 16)] = negv
        return 0
    lax.fori_loop(0, (N_M1 - 560) // 16, _pad_m1, 0)

    def _zero_ob0(i, _):
        ob_v[pl.ds(i * 16, 16)] = zero16f
        return 0
    lax.fori_loop(0, 63, _zero_ob0, 0)

    @pl.when(wid < N_IMG)
    def _():
        pltpu.sync_copy(ob_v.at[pl.ds(0, TOP_K * 5)], out_hbm.at[wid, 0])

    def ext_phase(carry, s_x, m1_x):
        m, w0, w1, w2 = carry
        s2, h2 = lastfind(w2, m, 32)
        s1, h1 = lastfind(w1, m, 16)
        s0, _ = lastfind(w0, m, 0)
        sci = jnp.where(h2, s2, jnp.where(h1, s1, s0))
        mv = plsc.load_gather(m1_x, [sci * 16 + iota])
        ci, _ = lastfind(mv, m, sci * 16)
        ev = plsc.load_gather(s_x, [ci * 16 + iota])
        li, _ = lastfind(ev, m, 0)
        gidx = ci * 16 + li
        ev2 = jnp.where(iota == li, NEG, ev)
        nm1 = treemax(ev2)
        mvo = jnp.where(iota == (ci - sci * 16), NEG, mv)
        nm2 = jnp.maximum(treemax(mvo), nm1)
        upd = iota == (sci & 15)
        wsel = sci >> 4
        u0 = (wsel == 0) & upd
        u1 = (wsel == 1) & upd
        u2 = (wsel == 2) & upd
        om = treemax(jnp.maximum(jnp.maximum(
            jnp.where(u0, NEG, w0), jnp.where(u1, NEG, w1)),
            jnp.where(u2, NEG, w2)))
        nm = jnp.maximum(om, nm2)
        n0 = jnp.where(u0, nm2, w0)
        n1 = jnp.where(u1, nm2, w1)
        n2 = jnp.where(u2, nm2, w2)
        return (nm, n0, n1, n2), (gidx, m, ci, nm1)

    def ext_stores(j, st, s_x, m1_x, si_x, ss_x):
        gidx, m, ci, nm1 = st
        jv = jnp.full((16,), j, jnp.int32)
        plsc.store_scatter(si_x, [jv], gidx, mask=mask0)
        plsc.store_scatter(ss_x, [jv], m, mask=mask0)
        plsc.store_scatter(s_x, [gidx], negv, mask=mask0)
        plsc.store_scatter(m1_x, [ci], nm1, mask=mask0)

    def init_carry(m2_x):
        w0 = m2_x[pl.ds(0, 16)]
        w1 = m2_x[pl.ds(16, 16)]
        w2 = m2_x[pl.ds(32, 16)]
        return treemax(jnp.maximum(jnp.maximum(w0, w1), w2)), w0, w1, w2

    def finish(cidx, si_x, ss_x):
        tmask = (iota + 192) >= TOP_K
        tsl = pl.ds(192, 16)
        si_x[tsl] = jnp.where(tmask, 0, si_x[tsl])
        ss_x[tsl] = jnp.where(tmask, -1.0, ss_x[tsl])
        si_x[pl.ds(208, 16)] = zero16i
        ss_x[pl.ds(208, 16)] = jnp.full((16,), -1.0, jnp.float32)

        def gsel(g, _):
            sl = pl.ds(g * 16, 16)
            idxv = si_x[sl]
            sx1_v[sl] = plsc.load_gather(bx1_v, [idxv])
            sy1_v[sl] = plsc.load_gather(by1_v, [idxv])
            sx2_v[sl] = plsc.load_gather(bx2_v, [idxv])
            sy2_v[sl] = plsc.load_gather(by2_v, [idxv])
            sar_v[sl] = plsc.load_gather(ar_v, [idxv])
            sup_v[sl] = zero16i
            return 0
        lax.fori_loop(0, 14, gsel, 0)

        def nms_body(j, _):
            jv = jnp.full((16,), j, jnp.int32)
            scj = plsc.load_gather(ss_x, [jv])
            suj = plsc.load_gather(sup_v, [jv])
            kv = ((scj > CONF_T) & (suj == 0)).astype(jnp.int32)

            @pl.when(kv[0] == 1)
            def _():
                x1p = plsc.load_gather(sx1_v, [jv])
                y1p = plsc.load_gather(sy1_v, [jv])
                x2p = plsc.load_gather(sx2_v, [jv])
                y2p = plsc.load_gather(sy2_v, [jv])
                arp = plsc.load_gather(sar_v, [jv])

                def iou_cond(sl, gbase):
                    tx1 = jnp.maximum(sx1_v[sl], x1p)
                    ty1 = jnp.maximum(sy1_v[sl], y1p)
                    tx2 = jnp.minimum(sx2_v[sl], x2p)
                    ty2 = jnp.minimum(sy2_v[sl], y2p)
                    w = jnp.maximum(tx2 - tx1, 0.0)
                    h = jnp.maximum(ty2 - ty1, 0.0)
                    inter = w * h
                    union = arp + (sar_v[sl] - inter)
                    iou = inter / union
                    return (iou > NMS_T) & ((iota + gbase) > j)

                def inner(g2, _):
                    g = g2 * 2
                    sl0 = pl.ds(g * 16, 16)
                    sl1 = pl.ds(g * 16 + 16, 16)
                    c0 = iou_cond(sl0, g * 16)
                    c1 = iou_cond(sl1, g * 16 + 16)
                    s0 = sup_v[sl0]
                    s1 = sup_v[sl1]
                    sup_v[sl0] = jnp.where(c0, 1, s0)
                    sup_v[sl1] = jnp.where(c1, 1, s1)
                    return 0
                lax.fori_loop(j // 32, 7, inner, 0)
            return 0
        lax.fori_loop(0, TOP_K, nms_body, 0)

        def zero_ob(i, _):
            ob_v[pl.ds(i * 16, 16)] = zero16f
            return 0
        lax.fori_loop(0, 63, zero_ob, 0)

        def out_body(g, off):
            sl = pl.ds(g * 16, 16)
            scv = ss_x[sl]
            keep = (scv > CONF_T) & (sup_v[sl] == 0)
            ki = keep.astype(jnp.int32)
            pos = off + jnp.cumsum(ki) - 1
            base5 = pos * 5
            plsc.store_scatter(ob_v, [base5], scv, mask=keep)
            plsc.store_scatter(ob_v, [base5 + 1], sx1_v[sl], mask=keep)
            plsc.store_scatter(ob_v, [base5 + 2], sy1_v[sl], mask=keep)
            plsc.store_scatter(ob_v, [base5 + 3], sx2_v[sl], mask=keep)
            plsc.store_scatter(ob_v, [base5 + 4], sy2_v[sl], mask=keep)
            return off + jnp.sum(ki)
        lax.fori_loop(0, 13, out_body, jnp.int32(0))

        pltpu.sync_copy(ob_v.at[pl.ds(0, TOP_K * 5)], out_hbm.at[b, cidx])

    def pair2(cA, cB):
        pltpu.sync_copy(scores_hbm.at[b, cA], sA.at[pl.ds(0, N_PAD)])
        pltpu.sync_copy(scores_hbm.at[b, cB], sB.at[pl.ds(0, N_PAD)])

        def b1(j, _):
            base = iota * 16 + j * 256
            aX = plsc.load_gather(sA, [base])
            aY = plsc.load_gather(sB, [base])
            for t in range(1, 16):
                aX = jnp.maximum(aX, plsc.load_gather(sA, [base + t]))
                aY = jnp.maximum(aY, plsc.load_gather(sB, [base + t]))
            m1A[pl.ds(j * 16, 16)] = aX
            m1B[pl.ds(j * 16, 16)] = aY
            return 0
        lax.fori_loop(0, 35, b1, 0)

        def b2(j, _):
            base = iota * 16 + j * 256
            aX = plsc.load_gather(m1A, [base])
            aY = plsc.load_gather(m1B, [base])
            for t in range(1, 16):
                aX = jnp.maximum(aX, plsc.load_gather(m1A, [base + t]))
                aY = jnp.maximum(aY, plsc.load_gather(m1B, [base + t]))
            m2A[pl.ds(j * 16, 16)] = aX
            m2B[pl.ds(j * 16, 16)] = aY
            return 0
        lax.fori_loop(0, 3, b2, 0)

        def ext2(j, c2):
            ncA, stA = ext_phase(c2[0], sA, m1A)
            ncB, stB = ext_phase(c2[1], sB, m1B)
            ext_stores(j, stA, sA, m1A, siA, ssA)
            ext_stores(j, stB, sB, m1B, siB, ssB)
            return ncA, ncB
        lax.fori_loop(0, TOP_K, ext2, (init_carry(m2A), init_carry(m2B)))

        finish(cA, siA, ssA)
        finish(cB, siB, ssB)

    def single(c):
        pltpu.sync_copy(scores_hbm.at[b, c], sA.at[pl.ds(0, N_PAD)])

        def b1(j, _):
            base = iota * 16 + j * 256
            aX = plsc.load_gather(sA, [base])
            for t in range(1, 16):
                aX = jnp.maximum(aX, plsc.load_gather(sA, [base + t]))
            m1A[pl.ds(j * 16, 16)] = aX
            return 0
        lax.fori_loop(0, 35, b1, 0)

        def b2(j, _):
            base = iota * 16 + j * 256
            aX = plsc.load_gather(m1A, [base])
            for t in range(1, 16):
                aX = jnp.maximum(aX, plsc.load_gather(m1A, [base + t]))
            m2A[pl.ds(j * 16, 16)] = aX
            return 0
        lax.fori_loop(0, 3, b2, 0)

        def ext1(j, carry):
            nc, st = ext_phase(carry, sA, m1A)
            ext_stores(j, st, sA, m1A, siA, ssA)
            return nc
        lax.fori_loop(0, TOP_K, ext1, init_carry(m2A))

        finish(c, siA, ssA)

    cbase = 1 + part * 5
    pair2(cbase, cbase + 1)
    pair2(cbase + 2, cbase + 3)
    single(cbase + 4)


def kernel(loc_data, conf_data, dbox_list):
    conf_t = jnp.transpose(conf_data, (0, 2, 1))
    loc_t = jnp.transpose(loc_data, (0, 2, 1))
    dbox_t = jnp.transpose(dbox_list, (1, 0))
    scores, boxes, area = _prep(conf_t, loc_t, dbox_t)
    out = _sc_nms(scores, boxes, area)
    return out.reshape(N_IMG, N_CLS, TOP_K, 5)
